# baseline (speedup 1.0000x reference)
import functools

import jax
import jax.numpy as jnp
from jax import lax
from jax.experimental import pallas as pl
from jax.experimental.pallas import tpu as pltpu
from jax.experimental.pallas import tpu_sc as plsc

N_NODES = 10000
N_EDGES = 320000
D = 128
N_CLASSES = 40

NC = 2
NS = 16
TILES = NC * NS
RPT = D // TILES
CK = 4096
NCH = 79
E_PAD = NCH * CK
N_PAD = 10240
IROWS = E_PAD // NS

_mesh = plsc.VectorSubcoreMesh(
    core_axis_name="c", subcore_axis_name="s", num_cores=NC, num_subcores=NS)


@functools.partial(
    pl.kernel,
    out_type=jax.ShapeDtypeStruct((TILES, RPT * N_PAD), jnp.float32),
    mesh=_mesh,
    compiler_params=pltpu.CompilerParams(needs_layout_passes=False),
    scratch_types=[
        pltpu.VMEM((RPT * N_PAD,), jnp.float32),
        pltpu.VMEM((RPT * N_PAD,), jnp.float32),
        pltpu.VMEM((CK,), jnp.int32),
        pltpu.VMEM((CK,), jnp.int32),
        pltpu.VMEM_SHARED((E_PAD,), jnp.int32),
        pltpu.SemaphoreType.DMA,
        pltpu.SemaphoreType.DMA,
    ],
)
def _agg_call(feat_hbm, idx_hbm, out_hbm, feat_v, acc_v, idx0_v, idx1_v, idx_sh, sem, sem1):
    c = lax.axis_index("c")
    s = lax.axis_index("s")
    w = c * NS + s

    cp_feat = pltpu.async_copy(feat_hbm.at[w], feat_v, sem)
    pltpu.sync_copy(idx_hbm.at[pl.ds(s * IROWS, IROWS)],
                    idx_sh.at[pl.ds(s * IROWS, IROWS)])

    zero = jnp.zeros((16,), jnp.float32)

    def _zcol(i, carry):
        acc_v[pl.ds(i * 16, 16)] = zero
        return carry

    lax.fori_loop(0, RPT * N_PAD // 16, _zcol, 0)
    cp_feat.wait()
    plsc.subcore_barrier()

    mask16 = jnp.full((16,), 0xFFFF, jnp.int32)
    sh16 = jnp.full((16,), 16, jnp.int32)
    rowoff = [jnp.full((16,), r * N_PAD, jnp.int32) for r in range(RPT)]

    def _run(idx_v):
        @plsc.parallel_loop(0, CK // 16, unroll=5)
        def _group(gi):
            v = idx_v[pl.ds(gi * 16, 16)]
            sv = lax.bitwise_and(v, mask16)
            dv = lax.shift_right_logical(v, sh16)
            for r in range(RPT):
                g = plsc.load_gather(feat_v, [sv + rowoff[r]])
                plsc.addupdate_scatter(acc_v, [dv + rowoff[r]], g)

    def _chunk(i, carry):
        ch = i * 2
        cp_a = pltpu.async_copy(idx_sh.at[pl.ds(ch * CK, CK)], idx0_v, sem)
        cp_b = pltpu.async_copy(idx_sh.at[pl.ds((ch + 1) * CK, CK)], idx1_v, sem1)
        cp_a.wait()
        _run(idx0_v)
        cp_b.wait()
        _run(idx1_v)
        return carry

    lax.fori_loop(0, NCH // 2, _chunk, 0)
    pltpu.sync_copy(idx_sh.at[pl.ds((NCH - 1) * CK, CK)], idx0_v)
    _run(idx0_v)

    pltpu.sync_copy(acc_v, out_hbm.at[w])


def _mlp_body(x_ref, p_ref, w1_ref, b1_ref, w2_ref, b2_ref, o_ref,
              *, final_relu):
    a = x_ref[...] + p_ref[...]
    t = jnp.dot(w1_ref[...], a, preferred_element_type=jnp.float32) + b1_ref[...]
    t = jnp.maximum(t, 0.0)
    o = jnp.dot(w2_ref[...], t, preferred_element_type=jnp.float32) + b2_ref[...]
    if final_relu:
        o = jnp.maximum(o, 0.0)
    o_ref[...] = o


def _mlp_call(xt, aggt, w1t, b1, w2t, b2, final_relu):
    bn = 1280
    grid = (N_PAD // bn,)
    col_spec = pl.BlockSpec((D, bn), lambda i: (0, i))
    full_spec = pl.BlockSpec((D, D), lambda i: (0, 0))
    bias_spec = pl.BlockSpec((D, 1), lambda i: (0, 0))
    return pl.pallas_call(
        functools.partial(_mlp_body, final_relu=final_relu),
        grid=grid,
        in_specs=[col_spec, col_spec, full_spec, bias_spec,
                  full_spec, bias_spec],
        out_specs=col_spec,
        out_shape=jax.ShapeDtypeStruct((D, N_PAD), jnp.float32),
    )(xt, aggt, w1t, b1.reshape(D, 1), w2t, b2.reshape(D, 1))


def kernel(x, edge_index, W1a, b1a, W2a, b2a, W1b, b1b, W2b, b2b):
    src = edge_index[0].astype(jnp.int32)
    dst = edge_index[1].astype(jnp.int32)
    pad = E_PAD - N_EDGES
    src_p = jnp.concatenate([src, jnp.zeros((pad,), jnp.int32)])
    dst_p = jnp.concatenate([dst, jnp.full((pad,), N_NODES, jnp.int32)])
    packed = jnp.bitwise_or(jnp.left_shift(dst_p, 16), src_p)

    xt = jnp.pad(x.T, ((0, 0), (0, N_PAD - N_NODES)))
    p = _agg_call(xt.reshape(TILES, RPT * N_PAD), packed)
    ht = _mlp_call(xt, p.reshape(D, N_PAD), W1a.T, b1a, W2a.T, b2a,
                   final_relu=True)

    p2 = _agg_call(ht.reshape(TILES, RPT * N_PAD), packed)
    w2bt = jnp.zeros((D, D), jnp.float32).at[:N_CLASSES, :].set(W2b.T)
    b2b_pad = jnp.zeros((D,), jnp.float32).at[:N_CLASSES].set(b2b)
    outt = _mlp_call(ht, p2.reshape(D, N_PAD), W1b.T, b1b, w2bt, b2b_pad,
                     final_relu=False)
    return outt[:N_CLASSES, :N_NODES].T


# speedup vs baseline: 1.0116x; 1.0116x over previous
import functools

import jax
import jax.numpy as jnp
from jax import lax
from jax.experimental import pallas as pl
from jax.experimental.pallas import tpu as pltpu
from jax.experimental.pallas import tpu_sc as plsc

N_NODES = 10000
N_EDGES = 320000
D = 128
N_CLASSES = 40

NC = 2
NS = 16
TILES = NC * NS
RPT = D // TILES
CK = 4096
NCH = 79
E_PAD = NCH * CK
N_PAD = 10240
IROWS = E_PAD // NS

_mesh = plsc.VectorSubcoreMesh(
    core_axis_name="c", subcore_axis_name="s", num_cores=NC, num_subcores=NS)


@functools.partial(
    pl.kernel,
    out_type=jax.ShapeDtypeStruct((TILES, RPT * N_PAD), jnp.float32),
    mesh=_mesh,
    compiler_params=pltpu.CompilerParams(needs_layout_passes=False),
    scratch_types=[
        pltpu.VMEM((RPT * N_PAD,), jnp.float32),
        pltpu.VMEM((RPT * N_PAD,), jnp.float32),
        pltpu.VMEM((CK,), jnp.int32),
        pltpu.VMEM((CK,), jnp.int32),
        pltpu.VMEM_SHARED((E_PAD,), jnp.int32),
        pltpu.SemaphoreType.DMA,
        pltpu.SemaphoreType.DMA,
    ],
)
def _agg_call(feat_hbm, idx_hbm, out_hbm, feat_v, acc_v, idx0_v, idx1_v, idx_sh, sem, sem1):
    c = lax.axis_index("c")
    s = lax.axis_index("s")
    w = c * NS + s

    cp_feat = pltpu.async_copy(feat_hbm.at[w], feat_v, sem)
    pltpu.sync_copy(idx_hbm.at[pl.ds(s * IROWS, IROWS)],
                    idx_sh.at[pl.ds(s * IROWS, IROWS)])

    zero = jnp.zeros((16,), jnp.float32)

    def _zcol(i, carry):
        acc_v[pl.ds(i * 16, 16)] = zero
        return carry

    lax.fori_loop(0, RPT * N_PAD // 16, _zcol, 0)
    cp_feat.wait()
    plsc.subcore_barrier()

    mask16 = jnp.full((16,), 0xFFFF, jnp.int32)
    sh16 = jnp.full((16,), 16, jnp.int32)
    rowoff = [jnp.full((16,), r * N_PAD, jnp.int32) for r in range(RPT)]

    def _run(idx_v):
        @plsc.parallel_loop(0, CK // 16, unroll=3)
        def _group(gi):
            v = idx_v[pl.ds(gi * 16, 16)]
            sv = lax.bitwise_and(v, mask16)
            dv = lax.shift_right_logical(v, sh16)
            for r in range(RPT):
                g = plsc.load_gather(feat_v, [sv + rowoff[r]])
                plsc.addupdate_scatter(acc_v, [dv + rowoff[r]], g)

    def _chunk(i, carry):
        ch = i * 2
        cp_a = pltpu.async_copy(idx_sh.at[pl.ds(ch * CK, CK)], idx0_v, sem)
        cp_b = pltpu.async_copy(idx_sh.at[pl.ds((ch + 1) * CK, CK)], idx1_v, sem1)
        cp_a.wait()
        _run(idx0_v)
        cp_b.wait()
        _run(idx1_v)
        return carry

    lax.fori_loop(0, NCH // 2, _chunk, 0)
    pltpu.sync_copy(idx_sh.at[pl.ds((NCH - 1) * CK, CK)], idx0_v)
    _run(idx0_v)

    pltpu.sync_copy(acc_v, out_hbm.at[w])


def _mlp_body(x_ref, p_ref, w1_ref, b1_ref, w2_ref, b2_ref, o_ref,
              *, final_relu):
    a = x_ref[...] + p_ref[...]
    t = jnp.dot(w1_ref[...], a, preferred_element_type=jnp.float32) + b1_ref[...]
    t = jnp.maximum(t, 0.0)
    o = jnp.dot(w2_ref[...], t, preferred_element_type=jnp.float32) + b2_ref[...]
    if final_relu:
        o = jnp.maximum(o, 0.0)
    o_ref[...] = o


def _mlp_call(xt, aggt, w1t, b1, w2t, b2, final_relu):
    bn = 1280
    grid = (N_PAD // bn,)
    col_spec = pl.BlockSpec((D, bn), lambda i: (0, i))
    full_spec = pl.BlockSpec((D, D), lambda i: (0, 0))
    bias_spec = pl.BlockSpec((D, 1), lambda i: (0, 0))
    return pl.pallas_call(
        functools.partial(_mlp_body, final_relu=final_relu),
        grid=grid,
        in_specs=[col_spec, col_spec, full_spec, bias_spec,
                  full_spec, bias_spec],
        out_specs=col_spec,
        out_shape=jax.ShapeDtypeStruct((D, N_PAD), jnp.float32),
    )(xt, aggt, w1t, b1.reshape(D, 1), w2t, b2.reshape(D, 1))


def kernel(x, edge_index, W1a, b1a, W2a, b2a, W1b, b1b, W2b, b2b):
    src = edge_index[0].astype(jnp.int32)
    dst = edge_index[1].astype(jnp.int32)
    pad = E_PAD - N_EDGES
    src_p = jnp.concatenate([src, jnp.zeros((pad,), jnp.int32)])
    dst_p = jnp.concatenate([dst, jnp.full((pad,), N_NODES, jnp.int32)])
    packed = jnp.bitwise_or(jnp.left_shift(dst_p, 16), src_p)

    xt = jnp.pad(x.T, ((0, 0), (0, N_PAD - N_NODES)))
    p = _agg_call(xt.reshape(TILES, RPT * N_PAD), packed)
    ht = _mlp_call(xt, p.reshape(D, N_PAD), W1a.T, b1a, W2a.T, b2a,
                   final_relu=True)

    p2 = _agg_call(ht.reshape(TILES, RPT * N_PAD), packed)
    w2bt = jnp.zeros((D, D), jnp.float32).at[:N_CLASSES, :].set(W2b.T)
    b2b_pad = jnp.zeros((D,), jnp.float32).at[:N_CLASSES].set(b2b)
    outt = _mlp_call(ht, p2.reshape(D, N_PAD), W1b.T, b1b, w2bt, b2b_pad,
                     final_relu=False)
    return outt[:N_CLASSES, :N_NODES].T
